# Initial kernel scaffold; baseline (speedup 1.0000x reference)
import jax
import jax.numpy as jnp
from jax.experimental import pallas as pl


def kernel(x, g, adj_unorm, W_lin, b_lin, W_mlp, b_mlp, W_str, b_str, W_a11, b_a11, W_a12, b_a12, W_a13, b_a13):
    raise NotImplementedError("write your pallas kernel here")


# speedup vs baseline: 1.0016x; 1.0016x over previous
import jax
import jax.numpy as jnp
from jax import lax
from jax.experimental import pallas as pl
from jax.experimental.pallas import tpu as pltpu

N = 10000
F = 128
BR = 200
NBLK = N // BR
BD = 1000


def _dotT(a, w):
    return lax.dot_general(a, w, (((1,), (1,)), ((), ())),
                           preferred_element_type=jnp.float32)


def _dot(a, b):
    return lax.dot_general(a, b, (((1,), (0,)), ((), ())),
                           preferred_element_type=jnp.float32)


def _pass1_kernel(x_ref, g_ref, adj_ref, wlin_ref, blin_ref, wmlp_ref,
                  bmlp_ref, wstrT_ref, bstr_ref,
                  h1_ref, y2_ref, y1s_ref, y2s_ref, h0_ref):
    i = pl.program_id(0)

    @pl.when(i == 0)
    def _init():
        h0 = jnp.maximum(_dotT(x_ref[...], wlin_ref[...]) + blin_ref[...], 0.0)
        h0_ref[...] = h0.astype(jnp.bfloat16)
        y1s_ref[...] = jnp.zeros_like(y1s_ref)
        y2s_ref[...] = jnp.zeros_like(y2s_ref)

    gb = g_ref[...].astype(jnp.bfloat16)
    h1_ref[...] = _dot(gb, h0_ref[...]).astype(jnp.bfloat16)

    ab = adj_ref[...].astype(jnp.bfloat16)
    y2 = jnp.maximum(_dot(ab, wstrT_ref[...]) + bstr_ref[...], 0.0)
    y2_ref[...] = y2
    y2s_ref[...] += jnp.sum(y2, axis=0, keepdims=True)

    base = pl.multiple_of(i * BR, 8)
    xb = x_ref[pl.ds(base, BR), :]
    y1b = jnp.maximum(_dotT(xb, wmlp_ref[...]) + bmlp_ref[...], 0.0)
    y1s_ref[...] += jnp.sum(y1b, axis=0, keepdims=True)


def _pass2_kernel(g_ref, h1_ref, h2_ref, h2s_ref):
    i = pl.program_id(0)

    @pl.when(i == 0)
    def _init():
        h2s_ref[...] = jnp.zeros_like(h2s_ref)

    gb = g_ref[...].astype(jnp.bfloat16)
    h2 = _dot(gb, h1_ref[...])
    h2_ref[...] = h2
    h2s_ref[...] += jnp.sum(h2, axis=0, keepdims=True)


def _leaky(v):
    return jnp.where(v >= 0.0, v, 0.01 * v)


def _pass3_kernel(x_ref, h2_ref, y2_ref, h2s_ref, y1s_ref, y2s_ref,
                  wmlp_ref, bmlp_ref, wa11_ref, ba11_ref, wa12_ref, ba12_ref,
                  wa13_ref, ba13_ref, out_ref):
    inv_n = 1.0 / N
    xm = h2s_ref[...] * inv_n
    y1m = y1s_ref[...] * inv_n
    y2m = y2s_ref[...] * inv_n

    def logit(w_ref, b_ref, va, vb):
        wa = w_ref[:, :F]
        wb = w_ref[:, F:]
        return (jnp.sum(va * wa, axis=1, keepdims=True)
                + jnp.sum(vb * wb, axis=1, keepdims=True) + b_ref[...])

    e11 = jnp.exp(_leaky(logit(wa11_ref, ba11_ref, xm, xm)))
    e12 = jnp.exp(_leaky(logit(wa12_ref, ba12_ref, xm, y1m)))
    e13 = jnp.exp(_leaky(logit(wa13_ref, ba13_ref, xm, y2m)))
    den = e11 + e12 + e13
    a11 = e11 / den
    a12 = e12 / den
    a13 = e13 / den

    y1b = jnp.maximum(_dotT(x_ref[...], wmlp_ref[...]) + bmlp_ref[...], 0.0)
    z = a11 * h2_ref[...] + a12 * y1b + a13 * y2_ref[...]
    out_ref[...] = jax.nn.sigmoid(z)


def kernel(x, g, adj_unorm, W_lin, b_lin, W_mlp, b_mlp, W_str, b_str,
           W_a11, b_a11, W_a12, b_a12, W_a13, b_a13):
    wstrT = W_str.T.astype(jnp.bfloat16)
    blin = b_lin.reshape(1, F)
    bmlp = b_mlp.reshape(1, F)
    bstr = b_str.reshape(1, F)
    ba11 = b_a11.reshape(1, 1)
    ba12 = b_a12.reshape(1, 1)
    ba13 = b_a13.reshape(1, 1)

    h1, y2, y1s, y2s = pl.pallas_call(
        _pass1_kernel,
        grid=(NBLK,),
        in_specs=[
            pl.BlockSpec((N, F), lambda i: (0, 0)),
            pl.BlockSpec((BR, N), lambda i: (i, 0)),
            pl.BlockSpec((BR, N), lambda i: (i, 0)),
            pl.BlockSpec((F, F), lambda i: (0, 0)),
            pl.BlockSpec((1, F), lambda i: (0, 0)),
            pl.BlockSpec((F, F), lambda i: (0, 0)),
            pl.BlockSpec((1, F), lambda i: (0, 0)),
            pl.BlockSpec((N, F), lambda i: (0, 0)),
            pl.BlockSpec((1, F), lambda i: (0, 0)),
        ],
        out_specs=[
            pl.BlockSpec((BR, F), lambda i: (i, 0)),
            pl.BlockSpec((BR, F), lambda i: (i, 0)),
            pl.BlockSpec((1, F), lambda i: (0, 0)),
            pl.BlockSpec((1, F), lambda i: (0, 0)),
        ],
        out_shape=[
            jax.ShapeDtypeStruct((N, F), jnp.bfloat16),
            jax.ShapeDtypeStruct((N, F), jnp.float32),
            jax.ShapeDtypeStruct((1, F), jnp.float32),
            jax.ShapeDtypeStruct((1, F), jnp.float32),
        ],
        scratch_shapes=[pltpu.VMEM((N, F), jnp.bfloat16)],
        compiler_params=pltpu.CompilerParams(
            dimension_semantics=("arbitrary",),
            vmem_limit_bytes=60 * 1024 * 1024,
        ),
    )(x, g, adj_unorm, W_lin, blin, W_mlp, bmlp, wstrT, bstr)

    h2, h2s = pl.pallas_call(
        _pass2_kernel,
        grid=(NBLK,),
        in_specs=[
            pl.BlockSpec((BR, N), lambda i: (i, 0)),
            pl.BlockSpec((N, F), lambda i: (0, 0)),
        ],
        out_specs=[
            pl.BlockSpec((BR, F), lambda i: (i, 0)),
            pl.BlockSpec((1, F), lambda i: (0, 0)),
        ],
        out_shape=[
            jax.ShapeDtypeStruct((N, F), jnp.float32),
            jax.ShapeDtypeStruct((1, F), jnp.float32),
        ],
        compiler_params=pltpu.CompilerParams(
            dimension_semantics=("arbitrary",),
            vmem_limit_bytes=60 * 1024 * 1024,
        ),
    )(g, h1)

    out = pl.pallas_call(
        _pass3_kernel,
        grid=(N // BD,),
        in_specs=[
            pl.BlockSpec((BD, F), lambda i: (i, 0)),
            pl.BlockSpec((BD, F), lambda i: (i, 0)),
            pl.BlockSpec((BD, F), lambda i: (i, 0)),
            pl.BlockSpec((1, F), lambda i: (0, 0)),
            pl.BlockSpec((1, F), lambda i: (0, 0)),
            pl.BlockSpec((1, F), lambda i: (0, 0)),
            pl.BlockSpec((F, F), lambda i: (0, 0)),
            pl.BlockSpec((1, F), lambda i: (0, 0)),
            pl.BlockSpec((1, 2 * F), lambda i: (0, 0)),
            pl.BlockSpec((1, 1), lambda i: (0, 0)),
            pl.BlockSpec((1, 2 * F), lambda i: (0, 0)),
            pl.BlockSpec((1, 1), lambda i: (0, 0)),
            pl.BlockSpec((1, 2 * F), lambda i: (0, 0)),
            pl.BlockSpec((1, 1), lambda i: (0, 0)),
        ],
        out_specs=pl.BlockSpec((BD, F), lambda i: (i, 0)),
        out_shape=jax.ShapeDtypeStruct((N, F), jnp.float32),
        compiler_params=pltpu.CompilerParams(
            dimension_semantics=("arbitrary",),
        ),
    )(x, h2, y2, h2s, y1s, y2s, W_mlp, bmlp, W_a11, ba11, W_a12, ba12,
      W_a13, ba13)
    return out
